# baseline (speedup 1.0000x reference)
import functools

import jax
import jax.numpy as jnp
from jax import lax
from jax.experimental import pallas as pl
from jax.experimental.pallas import tpu as pltpu
from jax.experimental.pallas import tpu_sc as plsc

N = 10000
E = 320000
D = 128
C = 8
H = 16
F = C * H

NC = 2
NS = 16
NW = NC * NS
CHUNK = 128
NCHUNK = 80
EDGES_PER_W = NCHUNK * CHUNK
EPAD = NW * EDGES_PER_W
PACK = 16384
ROW_STRIDE = 624
ROW_SPAN = 640
ZROWS = 128


def _sc_agg_body(h_hbm, packed_hbm, om_hbm, out_hbm,
                 packed_v, src_a, src_b, dst_a, dst_b, om_a, om_b,
                 rows_a, rows_b,
                 acc_sh, gsem_a, gsem_b, ssem_a, ssem_b, osem_a, osem_b):
    cid = lax.axis_index("c")
    sid = lax.axis_index("s")
    wid = cid * NS + sid

    pltpu.sync_copy(packed_hbm.at[wid], packed_v)

    def zbody(i, carry):
        r = i // C
        g = i - r * C
        rows_a[r, pl.ds(g * H, H)] = jnp.zeros((H,), jnp.float32)
        return carry
    lax.fori_loop(0, CHUNK * C, zbody, 0)
    for j in range(ROW_SPAN // CHUNK):
        pltpu.sync_copy(rows_a, acc_sh.at[pl.ds(sid * ROW_STRIDE + j * CHUNK, CHUNK)])
    plsc.subcore_barrier()

    nck = jnp.where(wid == NW - 1, (E - (NW - 1) * EDGES_PER_W) // CHUNK, NCHUNK)

    rows = (rows_a, rows_b)
    srcs = (src_a, src_b)
    dsts = (dst_a, dst_b)
    oms = (om_a, om_b)
    gsem = (gsem_a, gsem_b)
    ssem = (ssem_a, ssem_b)
    osem = (osem_a, osem_b)

    def unpack_idx(t, b):
        for g in range(CHUNK // H):
            v = packed_v[t, pl.ds(g * H, H)]
            srcs[b][pl.ds(g * H, H)] = v >> 14
            dsts[b][pl.ds(g * H, H)] = v & (PACK - 1)

    def compute_chunk(buf, om_v):
        @plsc.parallel_loop(0, CHUNK // 2, unroll=4)
        def pair_body(p):
            om16 = om_v[pl.ds(p * (2 * C), 2 * C)]
            e0 = p * 2
            for j in range(2 * C):
                e = e0 + j // C
                sl = pl.ds((j % C) * H, H)
                buf[e, sl] = buf[e, sl] * om16[j]

    def issue_fetch(t, b):
        base = (wid * EDGES_PER_W + t * CHUNK) * C
        pltpu.async_copy(om_hbm.at[pl.ds(base, CHUNK * C)], oms[b], osem[b])
        pltpu.async_copy(h_hbm.at[srcs[b]], rows[b], gsem[b])

    def drain_fetch(b):
        pltpu.make_async_copy(om_hbm.at[pl.ds(0, CHUNK * C)], oms[b], osem[b]).wait()
        pltpu.make_async_copy(h_hbm.at[srcs[b]], rows[b], gsem[b]).wait()

    unpack_idx(0, 0)
    issue_fetch(0, 0)

    def loop_body(t2, carry):
        for b in range(2):
            t = t2 * 2 + b
            nb = 1 - b

            @pl.when(t + 1 < nck)
            def _issue_next():
                @pl.when(t >= 1)
                def _drain_prev_scatter():
                    pltpu.make_async_copy(rows[nb], acc_sh.at[dsts[nb]],
                                          ssem[nb]).wait()
                unpack_idx(t + 1, nb)
                issue_fetch(t + 1, nb)

            drain_fetch(b)
            compute_chunk(rows[b], oms[b])
            pltpu.async_copy(rows[b], acc_sh.at[dsts[b]], ssem[b], add=True)
        return carry

    lax.fori_loop(0, nck // 2, loop_body, 0)

    pltpu.make_async_copy(rows[1], acc_sh.at[dsts[1]], ssem[1]).wait()
    pltpu.make_async_copy(rows[0], acc_sh.at[dsts[0]], ssem[0]).wait()
    plsc.subcore_barrier()

    for j in range(ROW_SPAN // ZROWS):
        r0 = sid * ROW_STRIDE + j * ZROWS
        pltpu.sync_copy(acc_sh.at[pl.ds(r0, ZROWS)],
                        out_hbm.at[cid, pl.ds(r0, ZROWS)])


_sc_agg = functools.partial(
    pl.kernel,
    out_type=jax.ShapeDtypeStruct((NC, N, F), jnp.float32),
    mesh=plsc.VectorSubcoreMesh(core_axis_name="c", subcore_axis_name="s",
                                num_cores=NC, num_subcores=NS),
    scratch_types=[
        pltpu.VMEM((NCHUNK, CHUNK), jnp.int32),
        pltpu.VMEM((CHUNK,), jnp.int32),
        pltpu.VMEM((CHUNK,), jnp.int32),
        pltpu.VMEM((CHUNK,), jnp.int32),
        pltpu.VMEM((CHUNK,), jnp.int32),
        pltpu.VMEM((CHUNK * C,), jnp.float32),
        pltpu.VMEM((CHUNK * C,), jnp.float32),
        pltpu.VMEM((CHUNK, F), jnp.float32),
        pltpu.VMEM((CHUNK, F), jnp.float32),
        pltpu.VMEM_SHARED((N, F), jnp.float32),
        pltpu.SemaphoreType.DMA,
        pltpu.SemaphoreType.DMA,
        pltpu.SemaphoreType.DMA,
        pltpu.SemaphoreType.DMA,
        pltpu.SemaphoreType.DMA,
        pltpu.SemaphoreType.DMA,
    ],
)(_sc_agg_body)


_BN = 1000


def _proj_body(x_ref, p_ref, o_ref):
    o_ref[...] = jnp.dot(x_ref[...], p_ref[...], preferred_element_type=jnp.float32)


def _post_body(parts_ref, wbd_ref, mavg_ref, g_ref, b_ref, o_ref, *, do_relu):
    s = parts_ref[0] + parts_ref[1]
    t = jnp.dot(s, wbd_ref[...], preferred_element_type=jnp.float32)
    mu = jnp.dot(t, mavg_ref[...], preferred_element_type=jnp.float32)
    d = t - mu
    var = jnp.dot(d * d, mavg_ref[...], preferred_element_type=jnp.float32)
    y = g_ref[...] * d * lax.rsqrt(var + 1e-5) + b_ref[...]
    if do_relu:
        y = jnp.maximum(y, 0.0)
    o_ref[...] = y


def _tc_proj(x, pflat):
    return pl.pallas_call(
        _proj_body,
        out_shape=jax.ShapeDtypeStruct((N, F), jnp.float32),
        grid=(N // _BN,),
        in_specs=[pl.BlockSpec((_BN, D), lambda i: (i, 0)),
                  pl.BlockSpec((D, F), lambda i: (0, 0))],
        out_specs=pl.BlockSpec((_BN, F), lambda i: (i, 0)),
    )(x, pflat)


def _tc_post(parts, wbd, mavg, gamma_t, beta_t, do_relu):
    return pl.pallas_call(
        functools.partial(_post_body, do_relu=do_relu),
        out_shape=jax.ShapeDtypeStruct((N, F), jnp.float32),
        grid=(N // _BN,),
        in_specs=[pl.BlockSpec((NC, _BN, F), lambda i: (0, i, 0)),
                  pl.BlockSpec((F, F), lambda i: (0, 0)),
                  pl.BlockSpec((F, F), lambda i: (0, 0)),
                  pl.BlockSpec((1, F), lambda i: (0, 0)),
                  pl.BlockSpec((1, F), lambda i: (0, 0))],
        out_specs=pl.BlockSpec((_BN, F), lambda i: (i, 0)),
    )(parts, wbd, mavg, gamma_t, beta_t)


def _blockdiag(w):
    eye = jnp.eye(C, dtype=w.dtype)
    return jnp.einsum('chk,cd->chdk', w, eye).reshape(C * H, C * w.shape[-1])


def kernel(x, edge_index, omega, proj0, w0, proj1, w1, ln_gamma, ln_beta):
    packed = jnp.pad(edge_index[0] * PACK + edge_index[1],
                     (0, EPAD - E)).reshape(NW, NCHUNK, CHUNK)
    om_flat = omega.reshape(E * C)

    mavg = jnp.kron(jnp.eye(C, dtype=jnp.float32),
                    jnp.full((H, H), 1.0 / H, dtype=jnp.float32))
    gamma_t = jnp.tile(ln_gamma, C).reshape(1, F)
    beta_t = jnp.tile(ln_beta, C).reshape(1, F)

    h0 = _tc_proj(x, proj0.reshape(D, F))
    parts1 = _sc_agg(h0, packed, om_flat)
    h1 = _tc_post(parts1, _blockdiag(w0), mavg, gamma_t, beta_t, True)
    parts2 = _sc_agg(h1, packed, om_flat)
    h2 = _tc_post(parts2, _blockdiag(w1), mavg, gamma_t, beta_t, False)
    return h2.reshape(N, C, H)


# speedup vs baseline: 1.0214x; 1.0214x over previous
import functools

import jax
import jax.numpy as jnp
from jax import lax
from jax.experimental import pallas as pl
from jax.experimental.pallas import tpu as pltpu
from jax.experimental.pallas import tpu_sc as plsc

N = 10000
E = 320000
D = 128
C = 8
H = 16
F = C * H

NC = 2
NS = 16
NW = NC * NS
CHUNK = 64
NCHUNK = 160
EDGES_PER_W = NCHUNK * CHUNK
EPAD = NW * EDGES_PER_W
PACK = 16384
ROW_STRIDE = 624
ROW_SPAN = 640
ZROWS = 128


def _sc_agg_body(h_hbm, packed_hbm, om_hbm, out_hbm,
                 packed_v, src_a, src_b, dst_a, dst_b, om_a, om_b,
                 rows_a, rows_b,
                 acc_sh, gsem_a, gsem_b, ssem_a, ssem_b, osem_a, osem_b):
    cid = lax.axis_index("c")
    sid = lax.axis_index("s")
    wid = cid * NS + sid

    pltpu.sync_copy(packed_hbm.at[wid], packed_v)

    def zbody(i, carry):
        r = i // C
        g = i - r * C
        rows_a[r, pl.ds(g * H, H)] = jnp.zeros((H,), jnp.float32)
        return carry
    lax.fori_loop(0, CHUNK * C, zbody, 0)
    for j in range(ROW_SPAN // CHUNK):
        pltpu.sync_copy(rows_a, acc_sh.at[pl.ds(sid * ROW_STRIDE + j * CHUNK, CHUNK)])
    plsc.subcore_barrier()

    nck = jnp.where(wid == NW - 1, (E - (NW - 1) * EDGES_PER_W) // CHUNK, NCHUNK)

    rows = (rows_a, rows_b)
    srcs = (src_a, src_b)
    dsts = (dst_a, dst_b)
    oms = (om_a, om_b)
    gsem = (gsem_a, gsem_b)
    ssem = (ssem_a, ssem_b)
    osem = (osem_a, osem_b)

    def unpack_idx(t, b):
        for g in range(CHUNK // H):
            v = packed_v[t, pl.ds(g * H, H)]
            srcs[b][pl.ds(g * H, H)] = v >> 14
            dsts[b][pl.ds(g * H, H)] = v & (PACK - 1)

    def compute_chunk(buf, om_v):
        @plsc.parallel_loop(0, CHUNK // 2, unroll=2)
        def pair_body(p):
            om16 = om_v[pl.ds(p * (2 * C), 2 * C)]
            e0 = p * 2
            for j in range(2 * C):
                e = e0 + j // C
                sl = pl.ds((j % C) * H, H)
                buf[e, sl] = buf[e, sl] * om16[j]

    def issue_fetch(t, b):
        base = (wid * EDGES_PER_W + t * CHUNK) * C
        pltpu.async_copy(om_hbm.at[pl.ds(base, CHUNK * C)], oms[b], osem[b])
        pltpu.async_copy(h_hbm.at[srcs[b]], rows[b], gsem[b])

    def drain_fetch(b):
        pltpu.make_async_copy(om_hbm.at[pl.ds(0, CHUNK * C)], oms[b], osem[b]).wait()
        pltpu.make_async_copy(h_hbm.at[srcs[b]], rows[b], gsem[b]).wait()

    unpack_idx(0, 0)
    issue_fetch(0, 0)

    def loop_body(t2, carry):
        for b in range(2):
            t = t2 * 2 + b
            nb = 1 - b

            @pl.when(t + 1 < nck)
            def _issue_next():
                @pl.when(t >= 1)
                def _drain_prev_scatter():
                    pltpu.make_async_copy(rows[nb], acc_sh.at[dsts[nb]],
                                          ssem[nb]).wait()
                unpack_idx(t + 1, nb)
                issue_fetch(t + 1, nb)

            drain_fetch(b)
            compute_chunk(rows[b], oms[b])
            pltpu.async_copy(rows[b], acc_sh.at[dsts[b]], ssem[b], add=True)
        return carry

    lax.fori_loop(0, nck // 2, loop_body, 0)

    pltpu.make_async_copy(rows[1], acc_sh.at[dsts[1]], ssem[1]).wait()
    pltpu.make_async_copy(rows[0], acc_sh.at[dsts[0]], ssem[0]).wait()
    plsc.subcore_barrier()

    for j in range(ROW_SPAN // ZROWS):
        r0 = sid * ROW_STRIDE + j * ZROWS
        pltpu.sync_copy(acc_sh.at[pl.ds(r0, ZROWS)],
                        out_hbm.at[cid, pl.ds(r0, ZROWS)])


_sc_agg = functools.partial(
    pl.kernel,
    out_type=jax.ShapeDtypeStruct((NC, N, F), jnp.float32),
    mesh=plsc.VectorSubcoreMesh(core_axis_name="c", subcore_axis_name="s",
                                num_cores=NC, num_subcores=NS),
    scratch_types=[
        pltpu.VMEM((NCHUNK, CHUNK), jnp.int32),
        pltpu.VMEM((CHUNK,), jnp.int32),
        pltpu.VMEM((CHUNK,), jnp.int32),
        pltpu.VMEM((CHUNK,), jnp.int32),
        pltpu.VMEM((CHUNK,), jnp.int32),
        pltpu.VMEM((CHUNK * C,), jnp.float32),
        pltpu.VMEM((CHUNK * C,), jnp.float32),
        pltpu.VMEM((CHUNK, F), jnp.float32),
        pltpu.VMEM((CHUNK, F), jnp.float32),
        pltpu.VMEM_SHARED((N, F), jnp.float32),
        pltpu.SemaphoreType.DMA,
        pltpu.SemaphoreType.DMA,
        pltpu.SemaphoreType.DMA,
        pltpu.SemaphoreType.DMA,
        pltpu.SemaphoreType.DMA,
        pltpu.SemaphoreType.DMA,
    ],
)(_sc_agg_body)


_BN = 1000


def _proj_body(x_ref, p_ref, o_ref):
    o_ref[...] = jnp.dot(x_ref[...], p_ref[...], preferred_element_type=jnp.float32)


def _post_body(parts_ref, wbd_ref, mavg_ref, g_ref, b_ref, o_ref, *, do_relu):
    s = parts_ref[0] + parts_ref[1]
    t = jnp.dot(s, wbd_ref[...], preferred_element_type=jnp.float32)
    mu = jnp.dot(t, mavg_ref[...], preferred_element_type=jnp.float32)
    d = t - mu
    var = jnp.dot(d * d, mavg_ref[...], preferred_element_type=jnp.float32)
    y = g_ref[...] * d * lax.rsqrt(var + 1e-5) + b_ref[...]
    if do_relu:
        y = jnp.maximum(y, 0.0)
    o_ref[...] = y


def _tc_proj(x, pflat):
    return pl.pallas_call(
        _proj_body,
        out_shape=jax.ShapeDtypeStruct((N, F), jnp.float32),
        grid=(N // _BN,),
        in_specs=[pl.BlockSpec((_BN, D), lambda i: (i, 0)),
                  pl.BlockSpec((D, F), lambda i: (0, 0))],
        out_specs=pl.BlockSpec((_BN, F), lambda i: (i, 0)),
    )(x, pflat)


def _tc_post(parts, wbd, mavg, gamma_t, beta_t, do_relu):
    return pl.pallas_call(
        functools.partial(_post_body, do_relu=do_relu),
        out_shape=jax.ShapeDtypeStruct((N, F), jnp.float32),
        grid=(N // _BN,),
        in_specs=[pl.BlockSpec((NC, _BN, F), lambda i: (0, i, 0)),
                  pl.BlockSpec((F, F), lambda i: (0, 0)),
                  pl.BlockSpec((F, F), lambda i: (0, 0)),
                  pl.BlockSpec((1, F), lambda i: (0, 0)),
                  pl.BlockSpec((1, F), lambda i: (0, 0))],
        out_specs=pl.BlockSpec((_BN, F), lambda i: (i, 0)),
    )(parts, wbd, mavg, gamma_t, beta_t)


def _blockdiag(w):
    eye = jnp.eye(C, dtype=w.dtype)
    return jnp.einsum('chk,cd->chdk', w, eye).reshape(C * H, C * w.shape[-1])


def kernel(x, edge_index, omega, proj0, w0, proj1, w1, ln_gamma, ln_beta):
    packed = jnp.pad(edge_index[0] * PACK + edge_index[1],
                     (0, EPAD - E)).reshape(NW, NCHUNK, CHUNK)
    om_flat = omega.reshape(E * C)

    mavg = jnp.kron(jnp.eye(C, dtype=jnp.float32),
                    jnp.full((H, H), 1.0 / H, dtype=jnp.float32))
    gamma_t = jnp.tile(ln_gamma, C).reshape(1, F)
    beta_t = jnp.tile(ln_beta, C).reshape(1, F)

    h0 = _tc_proj(x, proj0.reshape(D, F))
    parts1 = _sc_agg(h0, packed, om_flat)
    h1 = _tc_post(parts1, _blockdiag(w0), mavg, gamma_t, beta_t, True)
    parts2 = _sc_agg(h1, packed, om_flat)
    h2 = _tc_post(parts2, _blockdiag(w1), mavg, gamma_t, beta_t, False)
    return h2.reshape(N, C, H)


# speedup vs baseline: 1.1185x; 1.0951x over previous
import functools

import jax
import jax.numpy as jnp
from jax import lax
from jax.experimental import pallas as pl
from jax.experimental.pallas import tpu as pltpu
from jax.experimental.pallas import tpu_sc as plsc

N = 10000
E = 320000
D = 128
C = 8
H = 16
F = C * H

NC = 2
NS = 16
NW = NC * NS
CHUNK = 128
NCHUNK = 80
EDGES_PER_W = NCHUNK * CHUNK
EPAD = NW * EDGES_PER_W
PACK = 16384
ROW_STRIDE = 624
ROW_SPAN = 640
ZROWS = 128


def _sc_agg_body(h_hbm, packed_hbm, om_hbm, out_hbm,
                 packed_v, src_a, src_b, dst_a, dst_b, om_a, om_b,
                 rows_a, rows_b,
                 acc_sh, gsem_a, gsem_b, ssem_a, ssem_b, osem_a, osem_b):
    cid = lax.axis_index("c")
    sid = lax.axis_index("s")
    wid = cid * NS + sid

    pltpu.sync_copy(packed_hbm.at[wid], packed_v)

    def zbody(i, carry):
        r = i // C
        g = i - r * C
        rows_a[r, pl.ds(g * H, H)] = jnp.zeros((H,), jnp.float32)
        return carry
    lax.fori_loop(0, CHUNK * C, zbody, 0)
    for j in range(ROW_SPAN // CHUNK):
        pltpu.sync_copy(rows_a, acc_sh.at[pl.ds(sid * ROW_STRIDE + j * CHUNK, CHUNK)])
    plsc.subcore_barrier()

    nck = jnp.where(wid == NW - 1, (E - (NW - 1) * EDGES_PER_W) // CHUNK, NCHUNK)

    rows = (rows_a, rows_b)
    srcs = (src_a, src_b)
    dsts = (dst_a, dst_b)
    oms = (om_a, om_b)
    gsem = (gsem_a, gsem_b)
    ssem = (ssem_a, ssem_b)
    osem = (osem_a, osem_b)

    def unpack_idx(t, b):
        for g in range(CHUNK // H):
            v = packed_v[t, pl.ds(g * H, H)]
            srcs[b][pl.ds(g * H, H)] = v >> 14
            dsts[b][pl.ds(g * H, H)] = v & (PACK - 1)

    def compute_chunk(buf, om_v):
        @plsc.parallel_loop(0, CHUNK // 2, unroll=2)
        def pair_body(p):
            om16 = om_v[pl.ds(p * (2 * C), 2 * C)]
            e0 = p * 2
            for j in range(2 * C):
                e = e0 + j // C
                sl = pl.ds((j % C) * H, H)
                buf[e, sl] = buf[e, sl] * om16[j]

    def issue_fetch(t, b):
        base = (wid * EDGES_PER_W + t * CHUNK) * C
        pltpu.async_copy(om_hbm.at[pl.ds(base, CHUNK * C)], oms[b], osem[b])
        pltpu.async_copy(h_hbm.at[srcs[b]], rows[b], gsem[b])

    def drain_fetch(b):
        pltpu.make_async_copy(om_hbm.at[pl.ds(0, CHUNK * C)], oms[b], osem[b]).wait()
        pltpu.make_async_copy(h_hbm.at[srcs[b]], rows[b], gsem[b]).wait()

    unpack_idx(0, 0)
    issue_fetch(0, 0)

    def loop_body(t2, carry):
        for b in range(2):
            t = t2 * 2 + b
            nb = 1 - b

            @pl.when(t + 1 < nck)
            def _issue_next():
                @pl.when(t >= 1)
                def _drain_prev_scatter():
                    pltpu.make_async_copy(rows[nb], acc_sh.at[dsts[nb]],
                                          ssem[nb]).wait()
                unpack_idx(t + 1, nb)
                issue_fetch(t + 1, nb)

            drain_fetch(b)
            compute_chunk(rows[b], oms[b])
            pltpu.async_copy(rows[b], acc_sh.at[dsts[b]], ssem[b], add=True)
        return carry

    lax.fori_loop(0, nck // 2, loop_body, 0)

    pltpu.make_async_copy(rows[1], acc_sh.at[dsts[1]], ssem[1]).wait()
    pltpu.make_async_copy(rows[0], acc_sh.at[dsts[0]], ssem[0]).wait()
    plsc.subcore_barrier()

    for j in range(ROW_SPAN // ZROWS):
        r0 = sid * ROW_STRIDE + j * ZROWS
        pltpu.sync_copy(acc_sh.at[pl.ds(r0, ZROWS)],
                        out_hbm.at[cid, pl.ds(r0, ZROWS)])


_sc_agg = functools.partial(
    pl.kernel,
    out_type=jax.ShapeDtypeStruct((NC, N, F), jnp.float32),
    mesh=plsc.VectorSubcoreMesh(core_axis_name="c", subcore_axis_name="s",
                                num_cores=NC, num_subcores=NS),
    scratch_types=[
        pltpu.VMEM((NCHUNK, CHUNK), jnp.int32),
        pltpu.VMEM((CHUNK,), jnp.int32),
        pltpu.VMEM((CHUNK,), jnp.int32),
        pltpu.VMEM((CHUNK,), jnp.int32),
        pltpu.VMEM((CHUNK,), jnp.int32),
        pltpu.VMEM((CHUNK * C,), jnp.float32),
        pltpu.VMEM((CHUNK * C,), jnp.float32),
        pltpu.VMEM((CHUNK, F), jnp.float32),
        pltpu.VMEM((CHUNK, F), jnp.float32),
        pltpu.VMEM_SHARED((N, F), jnp.float32),
        pltpu.SemaphoreType.DMA,
        pltpu.SemaphoreType.DMA,
        pltpu.SemaphoreType.DMA,
        pltpu.SemaphoreType.DMA,
        pltpu.SemaphoreType.DMA,
        pltpu.SemaphoreType.DMA,
    ],
)(_sc_agg_body)


_BN = 1000


def _proj_body(x_ref, p_ref, e_ref, o_ref, pk_ref):
    o_ref[...] = jnp.dot(x_ref[...], p_ref[...], preferred_element_type=jnp.float32)
    pk_ref[...] = (e_ref[0] * PACK + e_ref[1]).reshape(pk_ref.shape)


def _post_body(parts_ref, wbd_ref, mavg_ref, g_ref, b_ref, o_ref, *, do_relu):
    s = parts_ref[0] + parts_ref[1]
    t = jnp.dot(s, wbd_ref[...], preferred_element_type=jnp.float32)
    mu = jnp.dot(t, mavg_ref[...], preferred_element_type=jnp.float32)
    d = t - mu
    var = jnp.dot(d * d, mavg_ref[...], preferred_element_type=jnp.float32)
    y = g_ref[...] * d * lax.rsqrt(var + 1e-5) + b_ref[...]
    if do_relu:
        y = jnp.maximum(y, 0.0)
    o_ref[...] = y


_EBLK = EPAD // (N // _BN)


def _tc_proj(x, pflat, edge_index):
    return pl.pallas_call(
        _proj_body,
        out_shape=(jax.ShapeDtypeStruct((N, F), jnp.float32),
                   jax.ShapeDtypeStruct((EPAD // CHUNK, CHUNK), jnp.int32)),
        grid=(N // _BN,),
        in_specs=[pl.BlockSpec((_BN, D), lambda i: (i, 0)),
                  pl.BlockSpec((D, F), lambda i: (0, 0)),
                  pl.BlockSpec((2, _EBLK), lambda i: (0, i))],
        out_specs=(pl.BlockSpec((_BN, F), lambda i: (i, 0)),
                   pl.BlockSpec((_EBLK // CHUNK, CHUNK), lambda i: (i, 0))),
    )(x, pflat, edge_index)


def _tc_post(parts, wbd, mavg, gamma_t, beta_t, do_relu):
    return pl.pallas_call(
        functools.partial(_post_body, do_relu=do_relu),
        out_shape=jax.ShapeDtypeStruct((N, F), jnp.float32),
        grid=(N // _BN,),
        in_specs=[pl.BlockSpec((NC, _BN, F), lambda i: (0, i, 0)),
                  pl.BlockSpec((F, F), lambda i: (0, 0)),
                  pl.BlockSpec((F, F), lambda i: (0, 0)),
                  pl.BlockSpec((1, F), lambda i: (0, 0)),
                  pl.BlockSpec((1, F), lambda i: (0, 0))],
        out_specs=pl.BlockSpec((_BN, F), lambda i: (i, 0)),
    )(parts, wbd, mavg, gamma_t, beta_t)


def _blockdiag(w):
    eye = jnp.eye(C, dtype=w.dtype)
    return jnp.einsum('chk,cd->chdk', w, eye).reshape(C * H, C * w.shape[-1])


def kernel(x, edge_index, omega, proj0, w0, proj1, w1, ln_gamma, ln_beta):
    om_flat = omega.reshape(E * C)

    mavg = jnp.kron(jnp.eye(C, dtype=jnp.float32),
                    jnp.full((H, H), 1.0 / H, dtype=jnp.float32))
    gamma_t = jnp.tile(ln_gamma, C).reshape(1, F)
    beta_t = jnp.tile(ln_beta, C).reshape(1, F)

    h0, packed2d = _tc_proj(x, proj0.reshape(D, F), edge_index)
    packed = packed2d.reshape(NW, NCHUNK, CHUNK)
    parts1 = _sc_agg(h0, packed, om_flat)
    h1 = _tc_post(parts1, _blockdiag(w0), mavg, gamma_t, beta_t, True)
    parts2 = _sc_agg(h1, packed, om_flat)
    h2 = _tc_post(parts2, _blockdiag(w1), mavg, gamma_t, beta_t, False)
    return h2.reshape(N, C, H)


# speedup vs baseline: 1.1192x; 1.0006x over previous
import functools

import jax
import jax.numpy as jnp
from jax import lax
from jax.experimental import pallas as pl
from jax.experimental.pallas import tpu as pltpu
from jax.experimental.pallas import tpu_sc as plsc

N = 10000
E = 320000
D = 128
C = 8
H = 16
F = C * H

NC = 2
NS = 16
NW = NC * NS
CHUNK = 128
NCHUNK = 80
EDGES_PER_W = NCHUNK * CHUNK
EPAD = NW * EDGES_PER_W
PACK = 16384
ROW_STRIDE = 624
ROW_SPAN = 640
ZROWS = 128


def _sc_agg_body(h_hbm, packed_hbm, om_hbm, out_hbm,
                 packed_v, src_a, src_b, dst_a, dst_b, om_a, om_b,
                 rows_a, rows_b,
                 acc_sh, gsem_a, gsem_b, ssem_a, ssem_b, osem_a, osem_b):
    cid = lax.axis_index("c")
    sid = lax.axis_index("s")
    wid = cid * NS + sid

    pltpu.sync_copy(packed_hbm.at[wid], packed_v)

    def zbody(i, carry):
        r = i // C
        g = i - r * C
        rows_a[r, pl.ds(g * H, H)] = jnp.zeros((H,), jnp.float32)
        return carry
    lax.fori_loop(0, CHUNK * C, zbody, 0)
    for j in range(ROW_SPAN // CHUNK):
        pltpu.sync_copy(rows_a, acc_sh.at[pl.ds(sid * ROW_STRIDE + j * CHUNK, CHUNK)])
    plsc.subcore_barrier()

    nck = jnp.where(wid == NW - 1, (E - (NW - 1) * EDGES_PER_W) // CHUNK, NCHUNK)

    rows = (rows_a, rows_b)
    srcs = (src_a, src_b)
    dsts = (dst_a, dst_b)
    oms = (om_a, om_b)
    gsem = (gsem_a, gsem_b)
    ssem = (ssem_a, ssem_b)
    osem = (osem_a, osem_b)

    def unpack_idx(t, b):
        for g in range(CHUNK // H):
            v = packed_v[t, pl.ds(g * H, H)]
            srcs[b][pl.ds(g * H, H)] = v >> 14
            dsts[b][pl.ds(g * H, H)] = v & (PACK - 1)

    def compute_chunk(buf, om_v):
        @plsc.parallel_loop(0, CHUNK // 2, unroll=2)
        def pair_body(p):
            om16 = om_v[p >> 3, pl.ds((p & 7) * (2 * C), 2 * C)]
            e0 = p * 2
            for j in range(2 * C):
                e = e0 + j // C
                sl = pl.ds((j % C) * H, H)
                buf[e, sl] = buf[e, sl] * om16[j]

    def issue_fetch(t, b):
        row0 = pl.multiple_of((wid * EDGES_PER_W + t * CHUNK) * C // F, 8)
        pltpu.async_copy(om_hbm.at[pl.ds(row0, CHUNK * C // F)], oms[b], osem[b])
        pltpu.async_copy(h_hbm.at[srcs[b]], rows[b], gsem[b])

    def drain_fetch(b):
        pltpu.make_async_copy(om_hbm.at[pl.ds(0, CHUNK * C // F)], oms[b], osem[b]).wait()
        pltpu.make_async_copy(h_hbm.at[srcs[b]], rows[b], gsem[b]).wait()

    unpack_idx(0, 0)
    issue_fetch(0, 0)

    def loop_body(t2, carry):
        for b in range(2):
            t = t2 * 2 + b
            nb = 1 - b

            @pl.when(t + 1 < nck)
            def _issue_next():
                @pl.when(t >= 1)
                def _drain_prev_scatter():
                    pltpu.make_async_copy(rows[nb], acc_sh.at[dsts[nb]],
                                          ssem[nb]).wait()
                unpack_idx(t + 1, nb)
                issue_fetch(t + 1, nb)

            drain_fetch(b)
            compute_chunk(rows[b], oms[b])
            pltpu.async_copy(rows[b], acc_sh.at[dsts[b]], ssem[b], add=True)
        return carry

    lax.fori_loop(0, nck // 2, loop_body, 0)

    pltpu.make_async_copy(rows[1], acc_sh.at[dsts[1]], ssem[1]).wait()
    pltpu.make_async_copy(rows[0], acc_sh.at[dsts[0]], ssem[0]).wait()
    plsc.subcore_barrier()

    for j in range(ROW_SPAN // ZROWS):
        r0 = sid * ROW_STRIDE + j * ZROWS
        pltpu.sync_copy(acc_sh.at[pl.ds(r0, ZROWS)],
                        out_hbm.at[cid, pl.ds(r0, ZROWS)])


_sc_agg = functools.partial(
    pl.kernel,
    out_type=jax.ShapeDtypeStruct((NC, N, F), jnp.float32),
    mesh=plsc.VectorSubcoreMesh(core_axis_name="c", subcore_axis_name="s",
                                num_cores=NC, num_subcores=NS),
    scratch_types=[
        pltpu.VMEM((NCHUNK, CHUNK), jnp.int32),
        pltpu.VMEM((CHUNK,), jnp.int32),
        pltpu.VMEM((CHUNK,), jnp.int32),
        pltpu.VMEM((CHUNK,), jnp.int32),
        pltpu.VMEM((CHUNK,), jnp.int32),
        pltpu.VMEM((CHUNK * C // F, F), jnp.float32),
        pltpu.VMEM((CHUNK * C // F, F), jnp.float32),
        pltpu.VMEM((CHUNK, F), jnp.float32),
        pltpu.VMEM((CHUNK, F), jnp.float32),
        pltpu.VMEM_SHARED((N, F), jnp.float32),
        pltpu.SemaphoreType.DMA,
        pltpu.SemaphoreType.DMA,
        pltpu.SemaphoreType.DMA,
        pltpu.SemaphoreType.DMA,
        pltpu.SemaphoreType.DMA,
        pltpu.SemaphoreType.DMA,
    ],
)(_sc_agg_body)


_BN = 1000


def _proj_body(x_ref, p_ref, e_ref, o_ref, pk_ref):
    o_ref[...] = jnp.dot(x_ref[...], p_ref[...], preferred_element_type=jnp.float32)
    pk_ref[...] = (e_ref[0] * PACK + e_ref[1]).reshape(pk_ref.shape)


def _post_body(parts_ref, wbd_ref, mavg_ref, g_ref, b_ref, o_ref, *, do_relu):
    s = parts_ref[0] + parts_ref[1]
    t = jnp.dot(s, wbd_ref[...], preferred_element_type=jnp.float32)
    mu = jnp.dot(t, mavg_ref[...], preferred_element_type=jnp.float32)
    d = t - mu
    var = jnp.dot(d * d, mavg_ref[...], preferred_element_type=jnp.float32)
    y = g_ref[...] * d * lax.rsqrt(var + 1e-5) + b_ref[...]
    if do_relu:
        y = jnp.maximum(y, 0.0)
    o_ref[...] = y


_EBLK = EPAD // (N // _BN)


def _tc_proj(x, pflat, edge_index):
    return pl.pallas_call(
        _proj_body,
        out_shape=(jax.ShapeDtypeStruct((N, F), jnp.float32),
                   jax.ShapeDtypeStruct((EPAD // CHUNK, CHUNK), jnp.int32)),
        grid=(N // _BN,),
        in_specs=[pl.BlockSpec((_BN, D), lambda i: (i, 0)),
                  pl.BlockSpec((D, F), lambda i: (0, 0)),
                  pl.BlockSpec((2, _EBLK), lambda i: (0, i))],
        out_specs=(pl.BlockSpec((_BN, F), lambda i: (i, 0)),
                   pl.BlockSpec((_EBLK // CHUNK, CHUNK), lambda i: (i, 0))),
    )(x, pflat, edge_index)


def _tc_post(parts, wbd, mavg, gamma_t, beta_t, do_relu):
    return pl.pallas_call(
        functools.partial(_post_body, do_relu=do_relu),
        out_shape=jax.ShapeDtypeStruct((N, F), jnp.float32),
        grid=(N // _BN,),
        in_specs=[pl.BlockSpec((NC, _BN, F), lambda i: (0, i, 0)),
                  pl.BlockSpec((F, F), lambda i: (0, 0)),
                  pl.BlockSpec((F, F), lambda i: (0, 0)),
                  pl.BlockSpec((1, F), lambda i: (0, 0)),
                  pl.BlockSpec((1, F), lambda i: (0, 0))],
        out_specs=pl.BlockSpec((_BN, F), lambda i: (i, 0)),
    )(parts, wbd, mavg, gamma_t, beta_t)


def _blockdiag(w):
    eye = jnp.eye(C, dtype=w.dtype)
    return jnp.einsum('chk,cd->chdk', w, eye).reshape(C * H, C * w.shape[-1])


def kernel(x, edge_index, omega, proj0, w0, proj1, w1, ln_gamma, ln_beta):
    om_flat = omega.reshape(E * C // F, F)

    mavg = jnp.kron(jnp.eye(C, dtype=jnp.float32),
                    jnp.full((H, H), 1.0 / H, dtype=jnp.float32))
    gamma_t = jnp.tile(ln_gamma, C).reshape(1, F)
    beta_t = jnp.tile(ln_beta, C).reshape(1, F)

    h0, packed2d = _tc_proj(x, proj0.reshape(D, F), edge_index)
    packed = packed2d.reshape(NW, NCHUNK, CHUNK)
    parts1 = _sc_agg(h0, packed, om_flat)
    h1 = _tc_post(parts1, _blockdiag(w0), mavg, gamma_t, beta_t, True)
    parts2 = _sc_agg(h1, packed, om_flat)
    h2 = _tc_post(parts2, _blockdiag(w1), mavg, gamma_t, beta_t, False)
    return h2.reshape(N, C, H)
